# baseline (speedup 1.0000x reference)
import functools

import jax
import jax.numpy as jnp
from jax import lax
from jax.experimental import pallas as pl
from jax.experimental.pallas import tpu as pltpu
from jax.experimental.pallas import tpu_sc as plsc

HIDDEN = 64
IDX_MINOR = 128
G = 4
CHUNK = IDX_MINOR * G
NWORKERS = 32


@jax.jit
def _sc_gather(ids_flat, table):
    n_total = ids_flat.shape[0]
    b_per_w = n_total // NWORKERS
    n_chunks = b_per_w // CHUNK

    mesh = plsc.VectorSubcoreMesh(core_axis_name="c", subcore_axis_name="s")

    @functools.partial(
        pl.kernel,
        mesh=mesh,
        out_type=jax.ShapeDtypeStruct((n_total, HIDDEN), jnp.float32),
        scratch_types=[
            pltpu.VMEM((CHUNK,), jnp.int32),
            pltpu.VMEM((CHUNK,), jnp.int32),
            pltpu.VMEM((CHUNK, HIDDEN), jnp.float32),
            pltpu.VMEM((CHUNK, HIDDEN), jnp.float32),
            pltpu.SemaphoreType.DMA,
            pltpu.SemaphoreType.DMA,
            pltpu.SemaphoreType.DMA,
            pltpu.SemaphoreType.DMA,
        ],
        compiler_params=pltpu.CompilerParams(use_tc_tiling_on_sc=False),
    )
    def k(ids_hbm, table_hbm, out_hbm, idx0, idx1, rows0, rows1,
          sg0, sg1, sw0, sw1):
        wid = lax.axis_index("s") * 2 + lax.axis_index("c")
        idx_v = (idx0, idx1)
        rows_v = (rows0, rows1)
        sem_g = (sg0, sg1)
        sem_w = (sw0, sw1)
        tok0 = wid * b_per_w

        def load_and_gather(c, b):
            pltpu.sync_copy(ids_hbm.at[pl.ds(tok0 + c * CHUNK, CHUNK)],
                            idx_v[b])
            for j in range(G):
                pltpu.async_copy(
                    table_hbm.at[idx_v[b].at[pl.ds(j * IDX_MINOR, IDX_MINOR)]],
                    rows_v[b].at[pl.ds(j * IDX_MINOR, IDX_MINOR)],
                    sem_g[b],
                )

        def gather_drain(b):
            pltpu.make_async_copy(
                out_hbm.at[pl.ds(0, CHUNK)], rows_v[b], sem_g[b]).wait()

        def write_start(c, b):
            pltpu.async_copy(
                rows_v[b],
                out_hbm.at[pl.ds(tok0 + c * CHUNK, CHUNK)],
                sem_w[b])

        def write_drain(b):
            pltpu.make_async_copy(
                out_hbm.at[pl.ds(0, CHUNK)], rows_v[b], sem_w[b]).wait()

        load_and_gather(0, 0)
        load_and_gather(1, 1)
        gather_drain(0)
        write_start(0, 0)

        def body(i, carry):
            for b in range(2):
                c = 2 * i + 2 + b
                write_drain(b)
                load_and_gather(c, b)
                gather_drain(1 - b)
                write_start(c - 1, 1 - b)
            return carry

        lax.fori_loop(0, (n_chunks - 2) // 2, body, 0)

        last_b = (n_chunks - 1) % 2
        gather_drain(last_b)
        write_start(n_chunks - 1, last_b)
        write_drain(1 - last_b)
        write_drain(last_b)

    return k(ids_flat, table)


def kernel(input_ids, token_embedding):
    batch, seq = input_ids.shape
    ids_flat = input_ids.reshape(batch * seq).astype(jnp.int32)
    rows = _sc_gather(ids_flat, token_embedding)
    return rows.reshape(batch, seq, HIDDEN)


# speedup vs baseline: 1.2259x; 1.2259x over previous
import functools

import jax
import jax.numpy as jnp
from jax import lax
from jax.experimental import pallas as pl
from jax.experimental.pallas import tpu as pltpu
from jax.experimental.pallas import tpu_sc as plsc

HIDDEN = 64
WIDE = 128
IDX_MINOR = 128
G = 2
CHUNK = IDX_MINOR * G
NWORKERS = 32


@jax.jit
def _sc_gather(ids_flat, table_wide):
    n_total = ids_flat.shape[0]
    b_per_w = n_total // NWORKERS
    n_chunks = b_per_w // CHUNK

    mesh = plsc.VectorSubcoreMesh(core_axis_name="c", subcore_axis_name="s")

    @functools.partial(
        pl.kernel,
        mesh=mesh,
        out_type=jax.ShapeDtypeStruct((n_total, WIDE), jnp.float32),
        scratch_types=[
            pltpu.VMEM((CHUNK,), jnp.int32),
            pltpu.VMEM((CHUNK,), jnp.int32),
            pltpu.VMEM((CHUNK, WIDE), jnp.float32),
            pltpu.VMEM((CHUNK, WIDE), jnp.float32),
            pltpu.SemaphoreType.DMA,
            pltpu.SemaphoreType.DMA,
            pltpu.SemaphoreType.DMA,
            pltpu.SemaphoreType.DMA,
        ],
        compiler_params=pltpu.CompilerParams(use_tc_tiling_on_sc=True),
    )
    def k(ids_hbm, table_hbm, out_hbm, idx0, idx1, rows0, rows1,
          sg0, sg1, sw0, sw1):
        wid = lax.axis_index("s") * 2 + lax.axis_index("c")
        idx_v = (idx0, idx1)
        rows_v = (rows0, rows1)
        sem_g = (sg0, sg1)
        sem_w = (sw0, sw1)
        tok0 = wid * b_per_w

        def load_and_gather(c, b):
            pltpu.sync_copy(ids_hbm.at[pl.ds(tok0 + c * CHUNK, CHUNK)],
                            idx_v[b])
            for j in range(G):
                pltpu.async_copy(
                    table_hbm.at[idx_v[b].at[pl.ds(j * IDX_MINOR, IDX_MINOR)]],
                    rows_v[b].at[pl.ds(j * IDX_MINOR, IDX_MINOR)],
                    sem_g[b],
                )

        def gather_drain(b):
            pltpu.make_async_copy(
                out_hbm.at[pl.ds(0, CHUNK)], rows_v[b], sem_g[b]).wait()

        def write_start(c, b):
            pltpu.async_copy(
                rows_v[b],
                out_hbm.at[pl.ds(tok0 + c * CHUNK, CHUNK)],
                sem_w[b])

        def write_drain(b):
            pltpu.make_async_copy(
                out_hbm.at[pl.ds(0, CHUNK)], rows_v[b], sem_w[b]).wait()

        load_and_gather(0, 0)
        load_and_gather(1, 1)
        gather_drain(0)
        write_start(0, 0)

        def body(i, carry):
            for b in range(2):
                c = 2 * i + 2 + b
                write_drain(b)
                load_and_gather(c, b)
                gather_drain(1 - b)
                write_start(c - 1, 1 - b)
            return carry

        lax.fori_loop(0, (n_chunks - 2) // 2, body, 0)

        last_b = (n_chunks - 1) % 2
        gather_drain(last_b)
        write_start(n_chunks - 1, last_b)
        write_drain(1 - last_b)
        write_drain(last_b)

    return k(ids_flat, table_wide)


def kernel(input_ids, token_embedding):
    batch, seq = input_ids.shape
    ids_flat = input_ids.reshape(batch * seq).astype(jnp.int32)
    table_wide = jnp.pad(token_embedding, ((0, 0), (0, WIDE - HIDDEN)))
    rows = _sc_gather(ids_flat, table_wide)
    return rows[:, :HIDDEN].reshape(batch, seq, HIDDEN)


# speedup vs baseline: 1.2289x; 1.0025x over previous
import functools

import jax
import jax.numpy as jnp
from jax import lax
from jax.experimental import pallas as pl
from jax.experimental.pallas import tpu as pltpu
from jax.experimental.pallas import tpu_sc as plsc

HIDDEN = 64
WIDE = 128
IDX_MINOR = 128
G = 2
CHUNK = IDX_MINOR * G
NWORKERS = 32


@jax.jit
def _sc_gather(ids_flat, table_wide):
    n_total = ids_flat.shape[0]
    b_per_w = n_total // NWORKERS
    n_chunks = b_per_w // CHUNK

    mesh = plsc.VectorSubcoreMesh(core_axis_name="c", subcore_axis_name="s")

    @functools.partial(
        pl.kernel,
        mesh=mesh,
        out_type=jax.ShapeDtypeStruct((n_total, WIDE), jnp.float32),
        scratch_types=[
            pltpu.VMEM((CHUNK,), jnp.int32),
            pltpu.VMEM((CHUNK,), jnp.int32),
            pltpu.VMEM((CHUNK, WIDE), jnp.float32),
            pltpu.VMEM((CHUNK, WIDE), jnp.float32),
            pltpu.SemaphoreType.DMA,
            pltpu.SemaphoreType.DMA,
            pltpu.SemaphoreType.DMA,
            pltpu.SemaphoreType.DMA,
        ],
        compiler_params=pltpu.CompilerParams(use_tc_tiling_on_sc=True),
    )
    def k(ids_hbm, table_hbm, out_hbm, idx0, idx1, rows0, rows1,
          sg0, sg1, sw0, sw1):
        wid = lax.axis_index("s") * 2 + lax.axis_index("c")
        idx_v = (idx0, idx1)
        rows_v = (rows0, rows1)
        sem_g = (sg0, sg1)
        sem_w = (sw0, sw1)
        tok0 = wid * b_per_w

        def load_and_gather(c, b):
            pltpu.sync_copy(ids_hbm.at[pl.ds(tok0 + c * CHUNK, CHUNK)],
                            idx_v[b])
            for j in range(G):
                pltpu.async_copy(
                    table_hbm.at[idx_v[b].at[pl.ds(j * IDX_MINOR, IDX_MINOR)]],
                    rows_v[b].at[pl.ds(j * IDX_MINOR, IDX_MINOR)],
                    sem_g[b],
                )

        def gather_drain(b):
            pltpu.make_async_copy(
                out_hbm.at[pl.ds(0, CHUNK)], rows_v[b], sem_g[b]).wait()

        def write_start(c, b):
            pltpu.async_copy(
                rows_v[b],
                out_hbm.at[pl.ds(tok0 + c * CHUNK, CHUNK)],
                sem_w[b])

        def write_drain(b):
            pltpu.make_async_copy(
                out_hbm.at[pl.ds(0, CHUNK)], rows_v[b], sem_w[b]).wait()

        load_and_gather(0, 0)
        load_and_gather(1, 1)
        gather_drain(0)
        write_start(0, 0)

        def body(i, carry):
            for b in range(2):
                c = 2 * i + 2 + b
                write_drain(b)
                load_and_gather(c, b)
                gather_drain(1 - b)
                write_start(c - 1, 1 - b)
            return carry

        lax.fori_loop(0, (n_chunks - 2) // 2, body, 0)

        last_b = (n_chunks - 1) % 2
        gather_drain(last_b)
        write_start(n_chunks - 1, last_b)
        write_drain(1 - last_b)
        write_drain(last_b)

    return k(ids_flat, table_wide)


def kernel(input_ids, token_embedding):
    batch, seq = input_ids.shape
    ids_flat = input_ids.reshape(batch * seq).astype(jnp.int32)
    table_wide = jnp.pad(token_embedding.T, ((0, WIDE - HIDDEN), (0, 0))).T
    rows = _sc_gather(ids_flat, table_wide)
    return rows[:, :HIDDEN].reshape(batch, seq, HIDDEN)


# speedup vs baseline: 1.2302x; 1.0011x over previous
import functools

import jax
import jax.numpy as jnp
from jax import lax
from jax.experimental import pallas as pl
from jax.experimental.pallas import tpu as pltpu
from jax.experimental.pallas import tpu_sc as plsc

HIDDEN = 64
WIDE = 128
IDX_MINOR = 128
G = 2
CHUNK = IDX_MINOR * G
NWORKERS = 32


@jax.jit
def _sc_gather(ids_flat, table_wide):
    n_total = ids_flat.shape[0]
    b_per_w = n_total // NWORKERS
    n_chunks = b_per_w // CHUNK

    mesh = plsc.VectorSubcoreMesh(core_axis_name="c", subcore_axis_name="s")

    @functools.partial(
        pl.kernel,
        mesh=mesh,
        out_type=jax.ShapeDtypeStruct((n_total, WIDE), jnp.float32),
        scratch_types=[
            pltpu.VMEM((CHUNK,), jnp.int32),
            pltpu.VMEM((CHUNK,), jnp.int32),
            pltpu.VMEM((CHUNK, WIDE), jnp.float32),
            pltpu.VMEM((CHUNK, WIDE), jnp.float32),
            pltpu.SemaphoreType.DMA,
            pltpu.SemaphoreType.DMA,
            pltpu.SemaphoreType.DMA,
            pltpu.SemaphoreType.DMA,
        ],
        compiler_params=pltpu.CompilerParams(use_tc_tiling_on_sc=True),
    )
    def k(ids_hbm, table_hbm, out_hbm, idx0, idx1, rows0, rows1,
          sg0, sg1, sw0, sw1):
        wid = lax.axis_index("s") * 2 + lax.axis_index("c")
        idx_v = (idx0, idx1)
        rows_v = (rows0, rows1)
        sem_g = (sg0, sg1)
        sem_w = (sw0, sw1)
        tok0 = wid * b_per_w

        def load_and_gather(c, b):
            pltpu.sync_copy(ids_hbm.at[pl.ds(tok0 + c * CHUNK, CHUNK)],
                            idx_v[b])
            for j in range(G):
                pltpu.async_copy(
                    table_hbm.at[idx_v[b].at[pl.ds(j * IDX_MINOR, IDX_MINOR)]],
                    rows_v[b].at[pl.ds(j * IDX_MINOR, IDX_MINOR)],
                    sem_g[b],
                )

        def gather_drain(b):
            pltpu.make_async_copy(
                out_hbm.at[pl.ds(0, CHUNK)], rows_v[b], sem_g[b]).wait()

        def write_start(c, b):
            pltpu.async_copy(
                rows_v[b],
                out_hbm.at[pl.ds(tok0 + c * CHUNK, CHUNK)],
                sem_w[b])

        def write_drain(b):
            pltpu.make_async_copy(
                out_hbm.at[pl.ds(0, CHUNK)], rows_v[b], sem_w[b]).wait()

        load_and_gather(0, 0)
        load_and_gather(1, 1)
        gather_drain(0)
        write_start(0, 0)

        def body(i, carry):
            for b in range(2):
                c = 2 * i + 2 + b
                write_drain(b)
                load_and_gather(c, b)
                gather_drain(1 - b)
                write_start(c - 1, 1 - b)
            return carry

        lax.fori_loop(0, (n_chunks - 2) // 2, body, 0)

        last_b = (n_chunks - 1) % 2
        gather_drain(last_b)
        write_start(n_chunks - 1, last_b)
        write_drain(1 - last_b)
        write_drain(last_b)

    return k(ids_flat, table_wide)


def kernel(input_ids, token_embedding):
    batch, seq = input_ids.shape
    ids_flat = input_ids.reshape(batch * seq).astype(jnp.int32)
    vocab = token_embedding.shape[0]
    table_wide = lax.dynamic_update_slice(
        jnp.zeros((vocab, WIDE), jnp.float32), token_embedding, (0, 0))
    rows = _sc_gather(ids_flat, table_wide)
    return rows[:, :HIDDEN].reshape(batch, seq, HIDDEN)


# speedup vs baseline: 1.2343x; 1.0033x over previous
import functools

import jax
import jax.numpy as jnp
from jax import lax
from jax.experimental import pallas as pl
from jax.experimental.pallas import tpu as pltpu
from jax.experimental.pallas import tpu_sc as plsc

HIDDEN = 64
WIDE = 128
IDX_MINOR = 128
G = 2
CHUNK = IDX_MINOR * G
NWORKERS = 32
NIDX = 4


@jax.jit
def _sc_gather(ids_flat, table_wide):
    n_total = ids_flat.shape[0]
    b_per_w = n_total // NWORKERS
    n_chunks = b_per_w // CHUNK

    mesh = plsc.VectorSubcoreMesh(core_axis_name="c", subcore_axis_name="s")

    @functools.partial(
        pl.kernel,
        mesh=mesh,
        out_type=jax.ShapeDtypeStruct((n_total, WIDE), jnp.float32),
        scratch_types=(
            [pltpu.VMEM((CHUNK,), jnp.int32) for _ in range(NIDX)]
            + [pltpu.VMEM((CHUNK, WIDE), jnp.float32) for _ in range(2)]
            + [pltpu.SemaphoreType.DMA for _ in range(NIDX + 4)]
        ),
        compiler_params=pltpu.CompilerParams(use_tc_tiling_on_sc=True),
    )
    def k(ids_hbm, table_hbm, out_hbm,
          idx0, idx1, idx2, idx3, rows0, rows1,
          si0, si1, si2, si3, sg0, sg1, sw0, sw1):
        wid = lax.axis_index("s") * 2 + lax.axis_index("c")
        idx_v = (idx0, idx1, idx2, idx3)
        sem_i = (si0, si1, si2, si3)
        rows_v = (rows0, rows1)
        sem_g = (sg0, sg1)
        sem_w = (sw0, sw1)
        tok0 = wid * b_per_w

        def idx_start(c, q):
            pltpu.async_copy(ids_hbm.at[pl.ds(tok0 + c * CHUNK, CHUNK)],
                             idx_v[q], sem_i[q])

        def idx_wait(q):
            pltpu.make_async_copy(ids_hbm.at[pl.ds(0, CHUNK)],
                                  idx_v[q], sem_i[q]).wait()

        def fire_gathers(q, b):
            for j in range(G):
                pltpu.async_copy(
                    table_hbm.at[idx_v[q].at[pl.ds(j * IDX_MINOR, IDX_MINOR)]],
                    rows_v[b].at[pl.ds(j * IDX_MINOR, IDX_MINOR)],
                    sem_g[b],
                )

        def gather_drain(b):
            pltpu.make_async_copy(
                out_hbm.at[pl.ds(0, CHUNK)], rows_v[b], sem_g[b]).wait()

        def write_start(c, b):
            pltpu.async_copy(
                rows_v[b],
                out_hbm.at[pl.ds(tok0 + c * CHUNK, CHUNK)],
                sem_w[b])

        def write_drain(b):
            pltpu.make_async_copy(
                out_hbm.at[pl.ds(0, CHUNK)], rows_v[b], sem_w[b]).wait()

        for q in range(NIDX):
            idx_start(q, q)
        idx_wait(0)
        fire_gathers(0, 0)
        idx_wait(1)
        fire_gathers(1, 1)
        gather_drain(0)
        write_start(0, 0)
        idx_start(4, 0)
        write_drain(0)
        idx_wait(2)
        fire_gathers(2, 0)
        gather_drain(1)
        write_start(1, 1)
        idx_start(5, 1)
        write_drain(1)
        idx_wait(3)
        fire_gathers(3, 1)
        gather_drain(0)
        write_start(2, 0)
        idx_start(6, 2)

        def body(i, carry):
            for b in range(NIDX):
                c = NIDX * i + 4 + b
                rb = b % 2
                write_drain(rb)
                idx_wait(b)
                fire_gathers(b, rb)
                gather_drain(1 - rb)

                @pl.when(c + 3 < n_chunks)
                def _():
                    idx_start(c + 3, (b + 3) % NIDX)

                write_start(c - 1, 1 - rb)
            return carry

        lax.fori_loop(0, (n_chunks - 4) // NIDX, body, 0)

        last_b = (n_chunks - 1) % 2
        gather_drain(last_b)
        write_start(n_chunks - 1, last_b)
        write_drain(1 - last_b)
        write_drain(last_b)

    return k(ids_flat, table_wide)


def kernel(input_ids, token_embedding):
    batch, seq = input_ids.shape
    ids_flat = input_ids.reshape(batch * seq).astype(jnp.int32)
    table_wide = jnp.pad(token_embedding, ((0, 0), (0, WIDE - HIDDEN)))
    rows = _sc_gather(ids_flat, table_wide)
    return rows[:, :HIDDEN].reshape(batch, seq, HIDDEN)
